# baseline (speedup 1.0000x reference)
import functools

import jax
import jax.numpy as jnp
from jax import lax
from jax.experimental import pallas as pl
from jax.experimental.pallas import tpu as pltpu
from jax.experimental.pallas import tpu_sc as plsc

BATCH = 4096
SEQ = 200
HIDDEN = 64

NC = 2
NS = 16
NW = NC * NS

PER_W = BATCH // NW
SPLIT = 104
NBUF = 8
NITER = PER_W // NBUF


_mesh = plsc.VectorSubcoreMesh(
    core_axis_name="c", subcore_axis_name="s", num_cores=NC, num_subcores=NS
)


@functools.partial(
    pl.kernel,
    out_type=jax.ShapeDtypeStruct((BATCH * SEQ, 128), jnp.float32),
    mesh=_mesh,
    compiler_params=pltpu.CompilerParams(use_tc_tiling_on_sc=False),
    scratch_types=[
        pltpu.VMEM((PER_W, SEQ), jnp.int32),
        pltpu.VMEM((NBUF, SEQ, HIDDEN), jnp.float32),
        pltpu.SemaphoreType.DMA,
        pltpu.SemaphoreType.DMA,
    ],
)
def _sc_gather(x_hbm, table_hbm, out_hbm, idx_v, bufs, sem_g, sem_s):
    wid = lax.axis_index("s") * NC + lax.axis_index("c")
    base = wid * PER_W

    pltpu.sync_copy(x_hbm.at[pl.ds(base, PER_W)], idx_v)

    def gather_a(j, b):
        return pltpu.make_async_copy(
            table_hbm.at[idx_v.at[j, pl.ds(0, SPLIT)]],
            bufs.at[b, pl.ds(0, SPLIT)], sem_g,
        )

    def gather_b(j, b):
        return pltpu.make_async_copy(
            table_hbm.at[idx_v.at[j, pl.ds(SPLIT, SEQ - SPLIT)]],
            bufs.at[b, pl.ds(SPLIT, SEQ - SPLIT)], sem_g,
        )

    def store_copy(j, b):
        return pltpu.make_async_copy(
            bufs.at[b],
            out_hbm.at[pl.ds((base + j) * SEQ, SEQ), pl.ds(0, HIDDEN)],
            sem_s,
        )

    for b in range(NBUF):
        gather_a(b, b).start()
        gather_b(b, b).start()

    def body(g, carry):
        j0 = g * NBUF
        for b in range(NBUF):
            gather_a(j0 + b, b).wait()
            gather_b(j0 + b, b).wait()
            store_copy(j0 + b, b).start()
        for b in range(NBUF):
            store_copy(j0 + b, b).wait()

            @pl.when(g + 1 < NITER)
            def _():
                gather_a(j0 + NBUF + b, b).start()
                gather_b(j0 + NBUF + b, b).start()

        return carry

    lax.fori_loop(0, NITER, body, 0)


def kernel(x, emb_table):
    wide = _sc_gather(x, emb_table)
    return wide.reshape(BATCH, SEQ, 128)[:, :, :HIDDEN]


# speedup vs baseline: 1.0010x; 1.0010x over previous
import functools

import jax
import jax.numpy as jnp
from jax import lax
from jax.experimental import pallas as pl
from jax.experimental.pallas import tpu as pltpu
from jax.experimental.pallas import tpu_sc as plsc

BATCH = 4096
SEQ = 200
HIDDEN = 64

NC = 2
NS = 16
NW = NC * NS

PER_W = BATCH // NW
SPLIT = 104
NBUF = 4
NITER = PER_W // NBUF


_mesh = plsc.VectorSubcoreMesh(
    core_axis_name="c", subcore_axis_name="s", num_cores=NC, num_subcores=NS
)


@functools.partial(
    pl.kernel,
    out_type=jax.ShapeDtypeStruct((BATCH * SEQ, 128), jnp.float32),
    mesh=_mesh,
    compiler_params=pltpu.CompilerParams(use_tc_tiling_on_sc=False),
    scratch_types=[
        pltpu.VMEM((PER_W, SEQ), jnp.int32),
        pltpu.VMEM((NBUF, SEQ, HIDDEN), jnp.float32),
        pltpu.SemaphoreType.DMA,
        pltpu.SemaphoreType.DMA,
    ],
)
def _sc_gather(x_hbm, table_hbm, out_hbm, idx_v, bufs, sem_g, sem_s):
    wid = lax.axis_index("s") * NC + lax.axis_index("c")
    base = wid * PER_W

    pltpu.sync_copy(x_hbm.at[pl.ds(base, PER_W)], idx_v)

    def gather_a(j, b):
        return pltpu.make_async_copy(
            table_hbm.at[idx_v.at[j, pl.ds(0, SPLIT)]],
            bufs.at[b, pl.ds(0, SPLIT)], sem_g,
        )

    def gather_b(j, b):
        return pltpu.make_async_copy(
            table_hbm.at[idx_v.at[j, pl.ds(SPLIT, SEQ - SPLIT)]],
            bufs.at[b, pl.ds(SPLIT, SEQ - SPLIT)], sem_g,
        )

    def store_copy(j, b):
        return pltpu.make_async_copy(
            bufs.at[b],
            out_hbm.at[pl.ds((base + j) * SEQ, SEQ), pl.ds(0, HIDDEN)],
            sem_s,
        )

    for b in range(NBUF):
        gather_a(b, b).start()
        gather_b(b, b).start()

    def body(g, carry):
        j0 = g * NBUF
        for b in range(NBUF):
            gather_a(j0 + b, b).wait()
            gather_b(j0 + b, b).wait()
            store_copy(j0 + b, b).start()
        for b in range(NBUF):
            store_copy(j0 + b, b).wait()

            @pl.when(g + 1 < NITER)
            def _():
                gather_a(j0 + NBUF + b, b).start()
                gather_b(j0 + NBUF + b, b).start()

        return carry

    lax.fori_loop(0, NITER, body, 0)


def kernel(x, emb_table):
    wide = _sc_gather(x, emb_table)
    return wide.reshape(BATCH, SEQ, 128)[:, :, :HIDDEN]
